# baseline (speedup 1.0000x reference)
import functools

import jax
import jax.numpy as jnp
from jax import lax
from jax.experimental import pallas as pl
from jax.experimental.pallas import tpu as pltpu
from jax.experimental.pallas import tpu_sc as plsc

EMBED_DIM = 64
CHUNK = 128
NBUF = 10

DOC_TOK = 4096 * 200
QRY_TOK = 4096 * 20

NW = 32
DOC_PER_W = DOC_TOK // NW
QRY_PER_W = QRY_TOK // NW
DOC_STEPS = DOC_PER_W // CHUNK
QRY_STEPS = QRY_PER_W // CHUNK
DOC_GROUPS = DOC_STEPS // NBUF
QRY_GROUPS = QRY_STEPS // NBUF


def _body(doc_idx, qry_idx, table, out_doc, out_qry,
          idx_d, idx_q, bufs, gsems):
    wid = lax.axis_index("s") * 2 + lax.axis_index("c")

    pltpu.sync_copy(doc_idx.at[pl.ds(wid * DOC_PER_W, DOC_PER_W)], idx_d)
    pltpu.sync_copy(qry_idx.at[pl.ds(wid * QRY_PER_W, QRY_PER_W)], idx_q)

    def run_phase(idx_ref, out_ref, base_tok, ngroups):
        def gather(c, b):
            pltpu.async_copy(
                table.at[idx_ref.at[pl.ds(c * CHUNK, CHUNK)]],
                bufs.at[b], gsems.at[b])

        def write(c, b):
            pltpu.sync_copy(
                bufs.at[b],
                out_ref.at[pl.ds(base_tok + c * CHUNK, CHUNK),
                           pl.ds(0, EMBED_DIM)])

        for b in range(NBUF):
            gather(b, b)

        def group_body(g, carry):
            base = g * NBUF
            for b in range(NBUF):
                pltpu.make_async_copy(
                    table.at[pl.ds(0, CHUNK)],
                    bufs.at[b], gsems.at[b]).wait()
                write(base + b, b)
            @pl.when(g < ngroups - 1)
            def _():
                for b in range(NBUF):
                    gather(base + NBUF + b, b)
            return carry

        lax.fori_loop(0, ngroups, group_body, 0)

    run_phase(idx_d, out_doc, wid * DOC_PER_W, DOC_GROUPS)
    run_phase(idx_q, out_qry, wid * QRY_PER_W, QRY_GROUPS)


@jax.jit
def _embed(doc_idx, qry_idx, table):
    mesh = plsc.VectorSubcoreMesh(core_axis_name="c", subcore_axis_name="s")
    run = functools.partial(
        pl.kernel,
        mesh=mesh,
        compiler_params=pltpu.CompilerParams(use_tc_tiling_on_sc=False,
                                             skip_device_barrier=True),
        out_type=[
            jax.ShapeDtypeStruct((DOC_TOK, 2 * EMBED_DIM), jnp.float32),
            jax.ShapeDtypeStruct((QRY_TOK, 2 * EMBED_DIM), jnp.float32),
        ],
        scratch_types=[
            pltpu.VMEM((DOC_PER_W,), jnp.int32),
            pltpu.VMEM((QRY_PER_W,), jnp.int32),
            pltpu.VMEM((NBUF, CHUNK, EMBED_DIM), jnp.float32),
            pltpu.SemaphoreType.DMA((NBUF,)),
        ],
    )(_body)
    return run(doc_idx, qry_idx, table)


def kernel(doc, qry, table):
    doc_idx = doc.reshape(DOC_TOK)
    qry_idx = qry.reshape(QRY_TOK)
    out_doc, out_qry = _embed(doc_idx, qry_idx, table)
    out_doc = out_doc.reshape(*doc.shape, 2 * EMBED_DIM)[:, :, :EMBED_DIM]
    out_qry = out_qry.reshape(*qry.shape, 2 * EMBED_DIM)[:, :, :EMBED_DIM]
    return (out_doc, out_qry)
